# Initial kernel scaffold; baseline (speedup 1.0000x reference)
import jax
import jax.numpy as jnp
from jax.experimental import pallas as pl


def kernel(var_node_features, con_node_features, edge_features, node_types, assoc_var, assoc_con, edge_index, edge_types, var_W1, var_b1, var_W2, var_b2, con_W1, con_b1, con_W2, con_b2, conv_basis, conv_att, conv_root, conv_bias, conv_h2v_W1, conv_h2v_b1, conv_h2v_W2, conv_h2v_b2, fc1_W, fc1_b, fc2_W, fc2_b, fc3_W, fc3_b, fc4_W, fc4_b):
    raise NotImplementedError("write your pallas kernel here")


# speedup vs baseline: 4.6885x; 4.6885x over previous
import functools

import jax
import jax.numpy as jnp
from jax import lax
from jax.experimental import pallas as pl
from jax.experimental.pallas import tpu as pltpu
from jax.experimental.pallas import tpu_sc as plsc

_HI = jax.lax.Precision.HIGHEST


def _mm(a, b):
  return jax.lax.dot(a, b, precision=_HI)


_D = 128
_NV = 5000
_N = 10000
_E = 160000
_NB = 5
_NCONV = 4
_BLK = 1000
_NBLK = _N // _BLK

_NC = 2
_NS = 16
_NW = _NC * _NS
_CH = 128
_CHUNKS = _E // _CH
_CPW = 40
_PAD_CHUNKS = _CPW * _NW - _CHUNKS
_ACC_ROWS = 10112
_RPT = _ACC_ROWS // _NS


def _rel_weights(att_ref, bm_ref, bl_ref, r):
  wf = att_ref[r, 0] * bm_ref[0]
  wl = att_ref[r, 0] * bl_ref[0]
  for b in range(1, _NB):
    wf = wf + att_ref[r, b] * bm_ref[b]
    wl = wl + att_ref[r, b] * bl_ref[b]
  return wf, wl


def _embed_body(x_ref, vw1, vb1, vw2, vb2, cw1, cb1, cw2, cb2, o_ref):
  isvar = pl.program_id(0) < _NV // _BLK
  w1 = jnp.where(isvar, vw1[...], cw1[...])
  b1 = jnp.where(isvar, vb1[...], cb1[...])
  w2 = jnp.where(isvar, vw2[...], cw2[...])
  b2 = jnp.where(isvar, vb2[...], cb2[...])
  h = jnp.maximum(_mm(x_ref[...], w1) + b1, 0.0)
  o_ref[...] = _mm(h, w2) + b2


def _gidx_body(src_ref, typ_ref, o_ref):
  o_ref[...] = src_ref[...] + _N * typ_ref[...]


def _table_body(xf_ref, xl_ref, bm_ref, bl_ref, att_ref, o_t):
  xf = xf_ref[...]
  xl = xl_ref[...]
  for r in range(2):
    wf, wl = _rel_weights(att_ref, bm_ref, bl_ref, r)
    o_t[r] = _mm(xf, wf) + xl * wl


def _update_table_body(p_ref, xf_ref, xl_ref, rff, rlrow, rcol, rll, bf, bl,
                       bm_ref, bl2_ref, att_ref, o_xf, o_xl, o_t):
  xf = xf_ref[...]
  xl = xl_ref[...]
  aggr = p_ref[0] + p_ref[1]
  nxf = jnp.maximum(aggr + _mm(xf, rff[...]) + xl * rlrow[...] + bf[...], 0.0)
  nxl = jnp.maximum(xl + _mm(xf, rcol[...]) + xl * rll[...] + bl[...], 0.0)
  o_xf[...] = nxf
  o_xl[...] = nxl
  for r in range(2):
    wf, wl = _rel_weights(att_ref, bm_ref, bl2_ref, r)
    o_t[r] = _mm(nxf, wf) + nxl * wl


def _update_last_body(p_ref, xf_ref, xl_ref, rff, rlrow, rcol, rll, bf, bl,
                      o_xf, o_xl):
  xf = xf_ref[...]
  xl = xl_ref[...]
  aggr = p_ref[0] + p_ref[1]
  o_xf[...] = jnp.maximum(aggr + _mm(xf, rff[...]) + xl * rlrow[...] + bf[...], 0.0)
  o_xl[...] = jnp.maximum(xl + _mm(xf, rcol[...]) + xl * rll[...] + bl[...], 0.0)


def _head_body(xf0, xf1, xf2, xf3, xf4, xl0, xl1, xl2, xl3, xl4,
               f1f0, f1f1, f1f2, f1f3, f1f4, f1l0, f1l1, f1l2, f1l3, f1l4,
               f1b, f2w, f2b, f3w, f3b, f4w, f4b, o_ref):
  xfs = (xf0, xf1, xf2, xf3, xf4)
  xls = (xl0, xl1, xl2, xl3, xl4)
  f1fs = (f1f0, f1f1, f1f2, f1f3, f1f4)
  f1ls = (f1l0, f1l1, f1l2, f1l3, f1l4)
  h = f1b[...]
  for i in range(5):
    h = h + _mm(xfs[i][...], f1fs[i][...]) + xls[i][...] * f1ls[i][...]
  h = jnp.maximum(h, 0.0)
  h = jnp.maximum(_mm(h, f2w[...]) + f2b[...], 0.0)
  h = jnp.maximum(_mm(h, f3w[...]) + f3b[...], 0.0)
  o_ref[...] = _mm(h, f4w[...]) + f4b[...]


def _full(arr_shape):
  nd = len(arr_shape)
  return pl.BlockSpec(arr_shape, lambda i, nd=nd: (0,) * nd)


def _rows(width):
  return pl.BlockSpec((_BLK, width), lambda i: (i, 0))


def _t_spec():
  return pl.BlockSpec((2, _BLK, _D), lambda i: (0, i, 0))


_SMEM_SPEC = pl.BlockSpec(memory_space=pltpu.SMEM)


_sc_mesh = plsc.VectorSubcoreMesh(
    core_axis_name="c", subcore_axis_name="s", num_cores=_NC, num_subcores=_NS)


@functools.partial(
    pl.kernel,
    out_type=jax.ShapeDtypeStruct((_NC, _ACC_ROWS, _D), jnp.float32),
    mesh=_sc_mesh,
    scratch_types=[
        pltpu.VMEM((_CPW, _CH), jnp.int32),
        pltpu.VMEM((_CPW, _CH), jnp.int32),
        pltpu.VMEM((_CH, _D), jnp.float32),
        pltpu.VMEM_SHARED((_ACC_ROWS, _D), jnp.float32),
        pltpu.SemaphoreType.DMA,
    ],
)
def _segsum(t_hbm, g_hbm, d_hbm, z_hbm, out_hbm, gi_v, di_v, rows_v, acc, sem):
  c = lax.axis_index("c")
  s = lax.axis_index("s")
  w = c * _NS + s
  pltpu.sync_copy(z_hbm, acc.at[pl.ds(s * _RPT, _RPT)])
  pltpu.sync_copy(g_hbm.at[pl.ds(w * _CPW, _CPW)], gi_v)
  pltpu.sync_copy(d_hbm.at[pl.ds(w * _CPW, _CPW)], di_v)
  plsc.subcore_barrier()

  def body(k, carry):
    pltpu.async_copy(t_hbm.at[gi_v.at[k]], rows_v, sem).wait()
    pltpu.sync_copy(rows_v, acc.at[di_v.at[k]], add=True)
    return carry

  lax.fori_loop(0, _CPW, body, 0)
  plsc.subcore_barrier()
  pltpu.sync_copy(acc.at[pl.ds(s * _RPT, _RPT)],
                  out_hbm.at[c, pl.ds(s * _RPT, _RPT)])


def kernel(var_node_features, con_node_features, edge_features, node_types,
           assoc_var, assoc_con, edge_index, edge_types,
           var_W1, var_b1, var_W2, var_b2, con_W1, con_b1, con_W2, con_b2,
           conv_basis, conv_att, conv_root, conv_bias,
           conv_h2v_W1, conv_h2v_b1, conv_h2v_W2, conv_h2v_b2,
           fc1_W, fc1_b, fc2_W, fc2_b, fc3_W, fc3_b, fc4_W, fc4_b):
  f32 = jnp.float32
  v = var_node_features.astype(f32)
  cfeat = con_node_features.astype(f32)
  xl0 = jnp.concatenate([v, cfeat], axis=0)
  src2d = edge_index[0].astype(jnp.int32).reshape(_CHUNKS, _CH)
  typ2d = edge_types.astype(jnp.int32).reshape(_CHUNKS, _CH)

  bmain = conv_basis[:, :, :_D, :]
  blast = conv_basis[:, :, _D:, :]

  xf0 = pl.pallas_call(
      _embed_body,
      grid=(_NBLK,),
      out_shape=jax.ShapeDtypeStruct((_N, _D), f32),
      in_specs=[_rows(1),
                _full((1, _D)), _full((1, _D)), _full((_D, _D)), _full((1, _D)),
                _full((1, _D)), _full((1, _D)), _full((_D, _D)), _full((1, _D))],
      out_specs=_rows(_D),
  )(xl0,
    var_W1, var_b1.reshape(1, _D), var_W2, var_b2.reshape(1, _D),
    con_W1, con_b1.reshape(1, _D), con_W2, con_b2.reshape(1, _D))

  g2d = pl.pallas_call(
      _gidx_body,
      out_shape=jax.ShapeDtypeStruct((_CHUNKS, _CH), jnp.int32),
  )(src2d, typ2d)

  def table_call(xf, xl, i):
    t3 = pl.pallas_call(
        _table_body,
        grid=(_NBLK,),
        out_shape=jax.ShapeDtypeStruct((2, _N, _D), f32),
        in_specs=[_rows(_D), _rows(1),
                  _full((_NB, _D, _D)), _full((_NB, 1, _D)), _SMEM_SPEC],
        out_specs=_t_spec(),
    )(xf, xl, bmain[i], blast[i], conv_att[i])
    return t3.reshape(2 * _N, _D)

  gpad = jnp.concatenate(
      [g2d, jnp.zeros((_PAD_CHUNKS, _CH), jnp.int32)], axis=0)
  dpad = jnp.concatenate(
      [edge_index[1].astype(jnp.int32).reshape(_CHUNKS, _CH),
       jnp.full((_PAD_CHUNKS, _CH), _N, jnp.int32)], axis=0)
  zrows = jnp.zeros((_RPT, _D), f32)

  t = table_call(xf0, xl0, 0)
  xfs, xls = [xf0], [xl0]
  xf, xl = xf0, xl0
  for i in range(_NCONV):
    partial = _segsum(t, gpad, dpad, zrows)
    root = conv_root[i]
    bias = conv_bias[i]
    args = (partial, xf, xl,
            root[:_D, :_D], root[_D:, :_D], root[:_D, _D:], root[_D:, _D:],
            bias[:_D].reshape(1, _D), bias[_D:].reshape(1, 1))
    wspecs = [_full((_D, _D)), _full((1, _D)), _full((_D, 1)), _full((1, 1)),
              _full((1, _D)), _full((1, 1))]
    if i < _NCONV - 1:
      xf, xl, t3 = pl.pallas_call(
          _update_table_body,
          grid=(_NBLK,),
          out_shape=(
              jax.ShapeDtypeStruct((_N, _D), f32),
              jax.ShapeDtypeStruct((_N, 1), f32),
              jax.ShapeDtypeStruct((2, _N, _D), f32),
          ),
          in_specs=[_t_spec(), _rows(_D), _rows(1)] + wspecs
                   + [_full((_NB, _D, _D)), _full((_NB, 1, _D)), _SMEM_SPEC],
          out_specs=(_rows(_D), _rows(1), _t_spec()),
      )(*args, bmain[i + 1], blast[i + 1], conv_att[i + 1])
      t = t3.reshape(2 * _N, _D)
    else:
      xf, xl = pl.pallas_call(
          _update_last_body,
          grid=(_NBLK,),
          out_shape=(
              jax.ShapeDtypeStruct((_N, _D), f32),
              jax.ShapeDtypeStruct((_N, 1), f32),
          ),
          in_specs=[_t_spec(), _rows(_D), _rows(1)] + wspecs,
          out_specs=(_rows(_D), _rows(1)),
      )(*args)
    xfs.append(xf)
    xls.append(xl)

  f1fs = [fc1_W[129 * i:129 * i + _D] for i in range(5)]
  f1ls = [fc1_W[129 * i + _D:129 * (i + 1)] for i in range(5)]
  out = pl.pallas_call(
      _head_body,
      grid=(_NV // _BLK,),
      out_shape=jax.ShapeDtypeStruct((_NV, 1), f32),
      in_specs=[_rows(_D)] * 5 + [_rows(1)] * 5
               + [_full((_D, _D))] * 5 + [_full((1, _D))] * 5
               + [_full((1, _D)), _full((_D, _D)), _full((1, _D)),
                  _full((_D, _D)), _full((1, _D)), _full((_D, 1)), _full((1, 1))],
      out_specs=_rows(1),
  )(*xfs, *xls, *f1fs, *f1ls,
    fc1_b.reshape(1, _D), fc2_W, fc2_b.reshape(1, _D),
    fc3_W, fc3_b.reshape(1, _D), fc4_W, fc4_b.reshape(1, 1))
  return out.reshape(_NV)


# speedup vs baseline: 6.2204x; 1.3268x over previous
import functools

import jax
import jax.numpy as jnp
from jax import lax
from jax.experimental import pallas as pl
from jax.experimental.pallas import tpu as pltpu
from jax.experimental.pallas import tpu_sc as plsc

_HI = jax.lax.Precision.HIGHEST


def _r16(x):
  return x.astype(jnp.bfloat16).astype(jnp.float32)


def _mm(a, b):
  return jax.lax.dot(a, b)


_D = 128
_DF = 129
_NV = 5000
_N = 10000
_E = 160000
_NB = 5
_NCONV = 4
_BLK = 1000
_NBLK = _N // _BLK

_NC = 2
_NS = 16
_NW = _NC * _NS
_CH = 128
_CHUNKS = _E // _CH
_CPW = 40
_PAD_CHUNKS = _CPW * _NW - _CHUNKS
_ACC_ROWS = 10112
_RPT = _ACC_ROWS // _NS


def _rel_weight(att_ref, basis_ref, r):
  w = _r16(att_ref[r, 0]) * _r16(basis_ref[0])
  for b in range(1, _NB):
    w = w + _r16(att_ref[r, b]) * _r16(basis_ref[b])
  return w


def _embed_body(x_ref, vw1, vb1, vw2, vb2, cw1, cb1, cw2, cb2, o_ref):
  isvar = pl.program_id(0) < _NV // _BLK
  w1 = jnp.where(isvar, vw1[...], cw1[...])
  b1 = jnp.where(isvar, vb1[...], cb1[...])
  w2 = jnp.where(isvar, vw2[...], cw2[...])
  b2 = jnp.where(isvar, vb2[...], cb2[...])
  raw = x_ref[...]
  h = jnp.maximum(raw * w1 + b1, 0.0)
  o_ref[:, 0:_D] = _mm(h, w2) + b2
  o_ref[:, _D:_DF] = raw


def _gidx_body(src_ref, typ_ref, o_ref):
  o_ref[...] = src_ref[...] + _N * typ_ref[...]


def _table_body(x_ref, basis_ref, att_ref, o_t):
  x = x_ref[...]
  for r in range(2):
    o_t[r] = _mm(x, _rel_weight(att_ref, basis_ref, r))


def _update_table_body(p_ref, x_ref, root_ref, bias_ref,
                       basis_ref, att_ref, o_x, o_t):
  x = x_ref[...]
  aggr = jnp.concatenate(
      [p_ref[0] + p_ref[1], x[:, _D:_DF]], axis=-1)
  nx = jnp.maximum(aggr + _mm(x, root_ref[...]) + bias_ref[...], 0.0)
  o_x[...] = nx
  for r in range(2):
    o_t[r] = _mm(nx, _rel_weight(att_ref, basis_ref, r))


def _update_last_body(p_ref, x_ref, root_ref, bias_ref, o_x):
  x = x_ref[...]
  aggr = jnp.concatenate(
      [p_ref[0] + p_ref[1], x[:, _D:_DF]], axis=-1)
  o_x[...] = jnp.maximum(aggr + _mm(x, root_ref[...]) + bias_ref[...], 0.0)


def _head_body(x0, x1, x2, x3, x4, f1w, f1b, f2w, f2b, f3w, f3b, f4w, f4b,
               o_ref):
  h = jnp.concatenate(
      [x0[...], x1[...], x2[...], x3[...], x4[...]], axis=-1)
  h = jnp.maximum(_mm(h, f1w[...]) + f1b[...], 0.0)
  h = jnp.maximum(_mm(h, f2w[...]) + f2b[...], 0.0)
  h = jnp.maximum(_mm(h, f3w[...]) + f3b[...], 0.0)
  o_ref[...] = _mm(h, f4w[...]) + f4b[...]


def _full(arr_shape):
  nd = len(arr_shape)
  return pl.BlockSpec(arr_shape, lambda i, nd=nd: (0,) * nd)


def _rows(width):
  return pl.BlockSpec((_BLK, width), lambda i: (i, 0))


def _t_spec():
  return pl.BlockSpec((2, _BLK, _D), lambda i: (0, i, 0))


_SMEM_SPEC = pl.BlockSpec(memory_space=pltpu.SMEM)


_sc_mesh = plsc.VectorSubcoreMesh(
    core_axis_name="c", subcore_axis_name="s", num_cores=_NC, num_subcores=_NS)

_NBUF = 2


@functools.partial(
    pl.kernel,
    out_type=jax.ShapeDtypeStruct((_NC, _ACC_ROWS, _D), jnp.float32),
    mesh=_sc_mesh,
    scratch_types=[
        pltpu.VMEM((_CPW, _CH), jnp.int32),
        pltpu.VMEM((_CPW, _CH), jnp.int32),
        pltpu.VMEM((_NBUF, _CH, _D), jnp.float32),
        pltpu.VMEM_SHARED((_ACC_ROWS, _D), jnp.float32),
        [pltpu.SemaphoreType.DMA] * _NBUF,
        pltpu.SemaphoreType.DMA,
    ],
)
def _segsum(t_hbm, g_hbm, d_hbm, z_hbm, out_hbm, gi_v, di_v, rows_v, acc,
            gsems, zsem):
  c = lax.axis_index("c")
  s = lax.axis_index("s")
  w = c * _NS + s
  zd = pltpu.async_copy(z_hbm, acc.at[pl.ds(s * _RPT, _RPT)], zsem)
  pltpu.sync_copy(g_hbm.at[pl.ds(w * _CPW, _CPW)], gi_v)
  pltpu.sync_copy(d_hbm.at[pl.ds(w * _CPW, _CPW)], di_v)
  zd.wait()
  plsc.subcore_barrier()

  descs = []
  for b in range(_NBUF):
    descs.append(
        pltpu.async_copy(t_hbm.at[gi_v.at[b]], rows_v.at[b], gsems[b]))
  for kk in range(_CPW):
    b = kk % _NBUF
    descs[kk].wait()
    pltpu.sync_copy(rows_v.at[b], acc.at[di_v.at[kk]], add=True)
    if kk + _NBUF < _CPW:
      descs.append(
          pltpu.async_copy(t_hbm.at[gi_v.at[kk + _NBUF]], rows_v.at[b],
                           gsems[b]))
  plsc.subcore_barrier()
  pltpu.sync_copy(acc.at[pl.ds(s * _RPT, _RPT)],
                  out_hbm.at[c, pl.ds(s * _RPT, _RPT)])


def kernel(var_node_features, con_node_features, edge_features, node_types,
           assoc_var, assoc_con, edge_index, edge_types,
           var_W1, var_b1, var_W2, var_b2, con_W1, con_b1, con_W2, con_b2,
           conv_basis, conv_att, conv_root, conv_bias,
           conv_h2v_W1, conv_h2v_b1, conv_h2v_W2, conv_h2v_b2,
           fc1_W, fc1_b, fc2_W, fc2_b, fc3_W, fc3_b, fc4_W, fc4_b):
  f32 = jnp.float32
  v = var_node_features.astype(f32)
  cfeat = con_node_features.astype(f32)
  xraw = jnp.concatenate([v, cfeat], axis=0)
  src2d = edge_index[0].astype(jnp.int32).reshape(_CHUNKS, _CH)
  typ2d = edge_types.astype(jnp.int32).reshape(_CHUNKS, _CH)

  x = pl.pallas_call(
      _embed_body,
      grid=(_NBLK,),
      out_shape=jax.ShapeDtypeStruct((_N, _DF), f32),
      in_specs=[_rows(1),
                _full((1, _D)), _full((1, _D)), _full((_D, _D)), _full((1, _D)),
                _full((1, _D)), _full((1, _D)), _full((_D, _D)), _full((1, _D))],
      out_specs=_rows(_DF),
  )(xraw,
    var_W1, var_b1.reshape(1, _D), var_W2, var_b2.reshape(1, _D),
    con_W1, con_b1.reshape(1, _D), con_W2, con_b2.reshape(1, _D))

  g2d = pl.pallas_call(
      _gidx_body,
      out_shape=jax.ShapeDtypeStruct((_CHUNKS, _CH), jnp.int32),
  )(src2d, typ2d)

  def table_call(xc, i):
    t3 = pl.pallas_call(
        _table_body,
        grid=(_NBLK,),
        out_shape=jax.ShapeDtypeStruct((2, _N, _D), f32),
        in_specs=[_rows(_DF), _full((_NB, _DF, _D)), _SMEM_SPEC],
        out_specs=_t_spec(),
    )(xc, conv_basis[i], conv_att[i])
    return t3.reshape(2 * _N, _D)

  gpad = jnp.concatenate(
      [g2d, jnp.zeros((_PAD_CHUNKS, _CH), jnp.int32)], axis=0)
  dpad = jnp.concatenate(
      [edge_index[1].astype(jnp.int32).reshape(_CHUNKS, _CH),
       jnp.full((_PAD_CHUNKS, _CH), _N, jnp.int32)], axis=0)
  zrows = jnp.zeros((_RPT, _D), f32)

  t = table_call(x, 0)
  xs = [x]
  for i in range(_NCONV):
    partial = _segsum(t, gpad, dpad, zrows)
    args = (partial, x, conv_root[i], conv_bias[i].reshape(1, _DF))
    if i < _NCONV - 1:
      x, t3 = pl.pallas_call(
          _update_table_body,
          grid=(_NBLK,),
          out_shape=(
              jax.ShapeDtypeStruct((_N, _DF), f32),
              jax.ShapeDtypeStruct((2, _N, _D), f32),
          ),
          in_specs=[_t_spec(), _rows(_DF), _full((_DF, _DF)), _full((1, _DF)),
                    _full((_NB, _DF, _D)), _SMEM_SPEC],
          out_specs=(_rows(_DF), _t_spec()),
      )(*args, conv_basis[i + 1], conv_att[i + 1])
      t = t3.reshape(2 * _N, _D)
    else:
      x = pl.pallas_call(
          _update_last_body,
          grid=(_NBLK,),
          out_shape=jax.ShapeDtypeStruct((_N, _DF), f32),
          in_specs=[_t_spec(), _rows(_DF), _full((_DF, _DF)), _full((1, _DF))],
          out_specs=_rows(_DF),
      )(*args)
    xs.append(x)

  out = pl.pallas_call(
      _head_body,
      grid=(_NV // _BLK,),
      out_shape=jax.ShapeDtypeStruct((_NV, 1), f32),
      in_specs=[_rows(_DF)] * 5
               + [_full((5 * _DF, _D)), _full((1, _D)), _full((_D, _D)),
                  _full((1, _D)), _full((_D, _D)), _full((1, _D)),
                  _full((_D, 1)), _full((1, 1))],
      out_specs=_rows(1),
  )(*xs,
    fc1_W, fc1_b.reshape(1, _D), fc2_W, fc2_b.reshape(1, _D),
    fc3_W, fc3_b.reshape(1, _D), fc4_W, fc4_b.reshape(1, 1))
  return out.reshape(_NV)


# speedup vs baseline: 15.7013x; 2.5241x over previous
import functools

import jax
import jax.numpy as jnp
from jax import lax
from jax.experimental import pallas as pl
from jax.experimental.pallas import tpu as pltpu
from jax.experimental.pallas import tpu_sc as plsc

_HI = jax.lax.Precision.HIGHEST


def _r16(x):
  return x.astype(jnp.bfloat16).astype(jnp.float32)


def _mm(a, b):
  return jax.lax.dot(a, b)


_D = 128
_DF = 129
_NV = 5000
_N = 10000
_E = 160000
_NB = 5
_NCONV = 4
_BLK = 1000
_NBLK = _N // _BLK

_NC = 2
_NS = 16
_NW = _NC * _NS
_CH = 128
_CHUNKS = _E // _CH
_CPW = 40
_PAD_CHUNKS = _CPW * _NW - _CHUNKS
_ACC_ROWS = 10112
_RPT = _ACC_ROWS // _NS


def _rel_weight(att_ref, basis_ref, r):
  w = _r16(att_ref[r, 0]) * _r16(basis_ref[0])
  for b in range(1, _NB):
    w = w + _r16(att_ref[r, b]) * _r16(basis_ref[b])
  return w


def _embed_body(x_ref, vw1, vb1, vw2, vb2, cw1, cb1, cw2, cb2, o_ref):
  isvar = pl.program_id(0) < _NV // _BLK
  w1 = jnp.where(isvar, vw1[...], cw1[...])
  b1 = jnp.where(isvar, vb1[...], cb1[...])
  w2 = jnp.where(isvar, vw2[...], cw2[...])
  b2 = jnp.where(isvar, vb2[...], cb2[...])
  raw = x_ref[...]
  h = jnp.maximum(raw * w1 + b1, 0.0)
  o_ref[:, 0:_D] = _mm(h, w2) + b2
  o_ref[:, _D:_DF] = raw


def _gidx_body(src_ref, typ_ref, o_ref):
  o_ref[...] = src_ref[...] + _N * typ_ref[...]


def _table_body(x_ref, basis_ref, att_ref, o_t):
  x = x_ref[...]
  for r in range(2):
    o_t[r] = _mm(x, _rel_weight(att_ref, basis_ref, r))


def _update_table_body(p_ref, x_ref, root_ref, bias_ref,
                       basis_ref, att_ref, o_x, o_t):
  x = x_ref[...]
  aggr = jnp.concatenate(
      [p_ref[0] + p_ref[1], x[:, _D:_DF]], axis=-1)
  nx = jnp.maximum(aggr + _mm(x, root_ref[...]) + bias_ref[...], 0.0)
  o_x[...] = nx
  for r in range(2):
    o_t[r] = _mm(nx, _rel_weight(att_ref, basis_ref, r))


def _update_last_body(p_ref, x_ref, root_ref, bias_ref, o_x):
  x = x_ref[...]
  aggr = jnp.concatenate(
      [p_ref[0] + p_ref[1], x[:, _D:_DF]], axis=-1)
  o_x[...] = jnp.maximum(aggr + _mm(x, root_ref[...]) + bias_ref[...], 0.0)


def _head_body(x0, x1, x2, x3, x4, f1w, f1b, f2w, f2b, f3w, f3b, f4w, f4b,
               o_ref):
  h = jnp.concatenate(
      [x0[...], x1[...], x2[...], x3[...], x4[...]], axis=-1)
  h = jnp.maximum(_mm(h, f1w[...]) + f1b[...], 0.0)
  h = jnp.maximum(_mm(h, f2w[...]) + f2b[...], 0.0)
  h = jnp.maximum(_mm(h, f3w[...]) + f3b[...], 0.0)
  o_ref[...] = _mm(h, f4w[...]) + f4b[...]


def _full(arr_shape):
  nd = len(arr_shape)
  return pl.BlockSpec(arr_shape, lambda i, nd=nd: (0,) * nd)


def _rows(width):
  return pl.BlockSpec((_BLK, width), lambda i: (i, 0))


def _t_spec():
  return pl.BlockSpec((2, _BLK, _D), lambda i: (0, i, 0))


_SMEM_SPEC = pl.BlockSpec(memory_space=pltpu.SMEM)


_sc_mesh = plsc.VectorSubcoreMesh(
    core_axis_name="c", subcore_axis_name="s", num_cores=_NC, num_subcores=_NS)

_NBUF = 2


@functools.partial(
    pl.kernel,
    out_type=jax.ShapeDtypeStruct((_NC, _ACC_ROWS, _D), jnp.float32),
    mesh=_sc_mesh,
    scratch_types=[
        pltpu.VMEM((_CPW, _CH), jnp.int32),
        pltpu.VMEM((_CPW, _CH), jnp.int32),
        pltpu.VMEM((_NBUF, _CH, _D), jnp.float32),
        pltpu.VMEM_SHARED((_ACC_ROWS, _D), jnp.float32),
        [pltpu.SemaphoreType.DMA] * _NBUF,
        pltpu.SemaphoreType.DMA,
    ],
)
def _segsum(t_hbm, g_hbm, d_hbm, z_hbm, out_hbm, gi_v, di_v, rows_v, acc,
            gsems, zsem):
  c = lax.axis_index("c")
  s = lax.axis_index("s")
  w = c * _NS + s
  zd = pltpu.async_copy(z_hbm, acc.at[pl.ds(s * _RPT, _RPT)], zsem)
  pltpu.sync_copy(g_hbm.at[pl.ds(w * _CPW, _CPW)], gi_v)
  pltpu.sync_copy(d_hbm.at[pl.ds(w * _CPW, _CPW)], di_v)
  zd.wait()
  plsc.subcore_barrier()

  descs = []
  for b in range(_NBUF):
    descs.append(
        pltpu.async_copy(t_hbm.at[gi_v.at[b]], rows_v.at[b], gsems[b]))
  for kk in range(_CPW):
    b = kk % _NBUF
    descs[kk].wait()
    pltpu.sync_copy(rows_v.at[b], acc.at[di_v.at[kk]], add=True)
    if kk + _NBUF < _CPW:
      descs.append(
          pltpu.async_copy(t_hbm.at[gi_v.at[kk + _NBUF]], rows_v.at[b],
                           gsems[b]))
  plsc.subcore_barrier()
  pltpu.sync_copy(acc.at[pl.ds(s * _RPT, _RPT)],
                  out_hbm.at[c, pl.ds(s * _RPT, _RPT)])


def kernel(var_node_features, con_node_features, edge_features, node_types,
           assoc_var, assoc_con, edge_index, edge_types,
           var_W1, var_b1, var_W2, var_b2, con_W1, con_b1, con_W2, con_b2,
           conv_basis, conv_att, conv_root, conv_bias,
           conv_h2v_W1, conv_h2v_b1, conv_h2v_W2, conv_h2v_b2,
           fc1_W, fc1_b, fc2_W, fc2_b, fc3_W, fc3_b, fc4_W, fc4_b):
  f32 = jnp.float32
  v = var_node_features.astype(f32)
  cfeat = con_node_features.astype(f32)
  xraw = jnp.concatenate([v, cfeat], axis=0)
  src2d = edge_index[0].astype(jnp.int32).reshape(_CHUNKS, _CH)
  typ2d = edge_types.astype(jnp.int32).reshape(_CHUNKS, _CH)

  x = pl.pallas_call(
      _embed_body,
      grid=(_NBLK,),
      out_shape=jax.ShapeDtypeStruct((_N, _DF), f32),
      in_specs=[_rows(1),
                _full((1, _D)), _full((1, _D)), _full((_D, _D)), _full((1, _D)),
                _full((1, _D)), _full((1, _D)), _full((_D, _D)), _full((1, _D))],
      out_specs=_rows(_DF),
  )(xraw,
    var_W1, var_b1.reshape(1, _D), var_W2, var_b2.reshape(1, _D),
    con_W1, con_b1.reshape(1, _D), con_W2, con_b2.reshape(1, _D))

  g2d = pl.pallas_call(
      _gidx_body,
      out_shape=jax.ShapeDtypeStruct((_CHUNKS, _CH), jnp.int32),
  )(src2d, typ2d)

  def table_call(xc, i):
    t3 = pl.pallas_call(
        _table_body,
        grid=(_NBLK,),
        out_shape=jax.ShapeDtypeStruct((2, _N, _D), f32),
        in_specs=[_rows(_DF), _full((_NB, _DF, _D)), _SMEM_SPEC],
        out_specs=_t_spec(),
    )(xc, conv_basis[i], conv_att[i])
    return t3.reshape(2 * _N, _D)

  ramp = jnp.arange(_PAD_CHUNKS * _CH, dtype=jnp.int32)
  gpad = jnp.concatenate(
      [g2d, (ramp % _N).reshape(_PAD_CHUNKS, _CH)], axis=0)
  dpad = jnp.concatenate(
      [edge_index[1].astype(jnp.int32).reshape(_CHUNKS, _CH),
       (_N + ramp % (_ACC_ROWS - _N)).reshape(_PAD_CHUNKS, _CH)],
      axis=0)
  zrows = jnp.zeros((_RPT, _D), f32)

  t = table_call(x, 0)
  xs = [x]
  for i in range(_NCONV):
    partial = _segsum(t, gpad, dpad, zrows)
    args = (partial, x, conv_root[i], conv_bias[i].reshape(1, _DF))
    if i < _NCONV - 1:
      x, t3 = pl.pallas_call(
          _update_table_body,
          grid=(_NBLK,),
          out_shape=(
              jax.ShapeDtypeStruct((_N, _DF), f32),
              jax.ShapeDtypeStruct((2, _N, _D), f32),
          ),
          in_specs=[_t_spec(), _rows(_DF), _full((_DF, _DF)), _full((1, _DF)),
                    _full((_NB, _DF, _D)), _SMEM_SPEC],
          out_specs=(_rows(_DF), _t_spec()),
      )(*args, conv_basis[i + 1], conv_att[i + 1])
      t = t3.reshape(2 * _N, _D)
    else:
      x = pl.pallas_call(
          _update_last_body,
          grid=(_NBLK,),
          out_shape=jax.ShapeDtypeStruct((_N, _DF), f32),
          in_specs=[_t_spec(), _rows(_DF), _full((_DF, _DF)), _full((1, _DF))],
          out_specs=_rows(_DF),
      )(*args)
    xs.append(x)

  out = pl.pallas_call(
      _head_body,
      grid=(_NV // _BLK,),
      out_shape=jax.ShapeDtypeStruct((_NV, 1), f32),
      in_specs=[_rows(_DF)] * 5
               + [_full((5 * _DF, _D)), _full((1, _D)), _full((_D, _D)),
                  _full((1, _D)), _full((_D, _D)), _full((1, _D)),
                  _full((_D, 1)), _full((1, 1))],
      out_specs=_rows(1),
  )(*xs,
    fc1_W, fc1_b.reshape(1, _D), fc2_W, fc2_b.reshape(1, _D),
    fc3_W, fc3_b.reshape(1, _D), fc4_W, fc4_b.reshape(1, 1))
  return out.reshape(_NV)


# speedup vs baseline: 16.6927x; 1.0631x over previous
import functools

import jax
import jax.numpy as jnp
from jax import lax
from jax.experimental import pallas as pl
from jax.experimental.pallas import tpu as pltpu
from jax.experimental.pallas import tpu_sc as plsc

_HI = jax.lax.Precision.HIGHEST


def _r16(x):
  return x.astype(jnp.bfloat16).astype(jnp.float32)


def _mm(a, b):
  return jax.lax.dot(a, b)


_D = 128
_DF = 129
_NV = 5000
_N = 10000
_E = 160000
_NB = 5
_NCONV = 4
_BLK = 1000
_NBLK = _N // _BLK

_NC = 2
_NS = 16
_NW = _NC * _NS
_CH = 128
_CHUNKS = _E // _CH
_CPW = 40
_PAD_CHUNKS = _CPW * _NW - _CHUNKS
_ACC_ROWS = 10112
_RPT = _ACC_ROWS // _NS


def _rel_weight(att_ref, basis_ref, r):
  w = _r16(att_ref[r, 0]) * _r16(basis_ref[0])
  for b in range(1, _NB):
    w = w + _r16(att_ref[r, b]) * _r16(basis_ref[b])
  return w


def _embed_body(x_ref, src_ref, typ_ref, vw1, vb1, vw2, vb2,
                cw1, cb1, cw2, cb2, basis_ref, att_ref, o_x, o_t, o_g):
  isvar = pl.program_id(0) < _NV // _BLK
  w1 = jnp.where(isvar, vw1[...], cw1[...])
  b1 = jnp.where(isvar, vb1[...], cb1[...])
  w2 = jnp.where(isvar, vw2[...], cw2[...])
  b2 = jnp.where(isvar, vb2[...], cb2[...])
  raw = x_ref[...]
  h = jnp.maximum(raw * w1 + b1, 0.0)
  x = jnp.concatenate([_mm(h, w2) + b2, raw], axis=-1)
  o_x[...] = x
  o_g[...] = src_ref[...] + _N * typ_ref[...]
  for r in range(2):
    o_t[r] = _mm(x, _rel_weight(att_ref, basis_ref, r))


def _update_table_body(p_ref, x_ref, root_ref, bias_ref,
                       basis_ref, att_ref, o_x, o_t):
  x = x_ref[...]
  aggr = jnp.concatenate(
      [p_ref[0] + p_ref[1], x[:, _D:_DF]], axis=-1)
  nx = jnp.maximum(aggr + _mm(x, root_ref[...]) + bias_ref[...], 0.0)
  o_x[...] = nx
  for r in range(2):
    o_t[r] = _mm(nx, _rel_weight(att_ref, basis_ref, r))


def _update_head_body(p_ref, x0, x1, x2, x3, root_ref, bias_ref,
                      f1w, f1b, f2w, f2b, f3w, f3b, f4w, f4b, o_ref):
  x = x3[...]
  aggr = jnp.concatenate(
      [p_ref[0] + p_ref[1], x[:, _D:_DF]], axis=-1)
  x4 = jnp.maximum(aggr + _mm(x, root_ref[...]) + bias_ref[...], 0.0)
  h = jnp.concatenate(
      [x0[...], x1[...], x2[...], x, x4], axis=-1)
  h = jnp.maximum(_mm(h, f1w[...]) + f1b[...], 0.0)
  h = jnp.maximum(_mm(h, f2w[...]) + f2b[...], 0.0)
  h = jnp.maximum(_mm(h, f3w[...]) + f3b[...], 0.0)
  o_ref[...] = _mm(h, f4w[...]) + f4b[...]


def _full(arr_shape):
  nd = len(arr_shape)
  return pl.BlockSpec(arr_shape, lambda i, nd=nd: (0,) * nd)


def _rows(width):
  return pl.BlockSpec((_BLK, width), lambda i: (i, 0))


def _t_spec():
  return pl.BlockSpec((2, _BLK, _D), lambda i: (0, i, 0))


_SMEM_SPEC = pl.BlockSpec(memory_space=pltpu.SMEM)


_sc_mesh = plsc.VectorSubcoreMesh(
    core_axis_name="c", subcore_axis_name="s", num_cores=_NC, num_subcores=_NS)

_NBUF = 2


@functools.partial(
    pl.kernel,
    out_type=jax.ShapeDtypeStruct((_NC, _ACC_ROWS, _D), jnp.float32),
    mesh=_sc_mesh,
    scratch_types=[
        pltpu.VMEM((_CPW, _CH), jnp.int32),
        pltpu.VMEM((_CPW, _CH), jnp.int32),
        pltpu.VMEM((_NBUF, _CH, _D), jnp.float32),
        pltpu.VMEM_SHARED((_ACC_ROWS, _D), jnp.float32),
        [pltpu.SemaphoreType.DMA] * _NBUF,
        pltpu.SemaphoreType.DMA,
    ],
)
def _segsum(t_hbm, g_hbm, d_hbm, z_hbm, out_hbm, gi_v, di_v, rows_v, acc,
            gsems, zsem):
  c = lax.axis_index("c")
  s = lax.axis_index("s")
  w = c * _NS + s
  zd = pltpu.async_copy(z_hbm, acc.at[pl.ds(s * _RPT, _RPT)], zsem)
  pltpu.sync_copy(g_hbm.at[pl.ds(w * _CPW, _CPW)], gi_v)
  pltpu.sync_copy(d_hbm.at[pl.ds(w * _CPW, _CPW)], di_v)
  zd.wait()
  plsc.subcore_barrier()

  descs = []
  for b in range(_NBUF):
    descs.append(
        pltpu.async_copy(t_hbm.at[gi_v.at[b]], rows_v.at[b], gsems[b]))
  for kk in range(_CPW):
    b = kk % _NBUF
    descs[kk].wait()
    pltpu.sync_copy(rows_v.at[b], acc.at[di_v.at[kk]], add=True)
    if kk + _NBUF < _CPW:
      descs.append(
          pltpu.async_copy(t_hbm.at[gi_v.at[kk + _NBUF]], rows_v.at[b],
                           gsems[b]))
  plsc.subcore_barrier()
  pltpu.sync_copy(acc.at[pl.ds(s * _RPT, _RPT)],
                  out_hbm.at[c, pl.ds(s * _RPT, _RPT)])


def kernel(var_node_features, con_node_features, edge_features, node_types,
           assoc_var, assoc_con, edge_index, edge_types,
           var_W1, var_b1, var_W2, var_b2, con_W1, con_b1, con_W2, con_b2,
           conv_basis, conv_att, conv_root, conv_bias,
           conv_h2v_W1, conv_h2v_b1, conv_h2v_W2, conv_h2v_b2,
           fc1_W, fc1_b, fc2_W, fc2_b, fc3_W, fc3_b, fc4_W, fc4_b):
  f32 = jnp.float32
  v = var_node_features.astype(f32)
  cfeat = con_node_features.astype(f32)
  xraw = jnp.concatenate([v, cfeat], axis=0)
  src2d = edge_index[0].astype(jnp.int32).reshape(_CHUNKS, _CH)
  typ2d = edge_types.astype(jnp.int32).reshape(_CHUNKS, _CH)

  gblk = _CHUNKS // _NBLK
  src3d = src2d.reshape(_NBLK, gblk, _CH)
  typ3d = typ2d.reshape(_NBLK, gblk, _CH)
  x, t3, g3d = pl.pallas_call(
      _embed_body,
      grid=(_NBLK,),
      out_shape=(
          jax.ShapeDtypeStruct((_N, _DF), f32),
          jax.ShapeDtypeStruct((2, _N, _D), f32),
          jax.ShapeDtypeStruct((_NBLK, gblk, _CH), jnp.int32),
      ),
      in_specs=[_rows(1),
                pl.BlockSpec((1, gblk, _CH), lambda i: (i, 0, 0)),
                pl.BlockSpec((1, gblk, _CH), lambda i: (i, 0, 0)),
                _full((1, _D)), _full((1, _D)), _full((_D, _D)), _full((1, _D)),
                _full((1, _D)), _full((1, _D)), _full((_D, _D)), _full((1, _D)),
                _full((_NB, _DF, _D)), _SMEM_SPEC],
      out_specs=(_rows(_DF), _t_spec(),
                 pl.BlockSpec((1, gblk, _CH), lambda i: (i, 0, 0))),
  )(xraw, src3d, typ3d,
    var_W1, var_b1.reshape(1, _D), var_W2, var_b2.reshape(1, _D),
    con_W1, con_b1.reshape(1, _D), con_W2, con_b2.reshape(1, _D),
    conv_basis[0], conv_att[0])
  t = t3.reshape(2 * _N, _D)
  g2d = g3d.reshape(_CHUNKS, _CH)

  ramp = jnp.arange(_PAD_CHUNKS * _CH, dtype=jnp.int32)
  gpad = jnp.concatenate(
      [g2d, (ramp % _N).reshape(_PAD_CHUNKS, _CH)], axis=0)
  dpad = jnp.concatenate(
      [edge_index[1].astype(jnp.int32).reshape(_CHUNKS, _CH),
       (_N + ramp % (_ACC_ROWS - _N)).reshape(_PAD_CHUNKS, _CH)],
      axis=0)
  zrows = jnp.zeros((_RPT, _D), f32)

  xs = [x]
  for i in range(_NCONV):
    partial = _segsum(t, gpad, dpad, zrows)
    args = (partial, x, conv_root[i], conv_bias[i].reshape(1, _DF))
    if i < _NCONV - 1:
      x, t3 = pl.pallas_call(
          _update_table_body,
          grid=(_NBLK,),
          out_shape=(
              jax.ShapeDtypeStruct((_N, _DF), f32),
              jax.ShapeDtypeStruct((2, _N, _D), f32),
          ),
          in_specs=[_t_spec(), _rows(_DF), _full((_DF, _DF)), _full((1, _DF)),
                    _full((_NB, _DF, _D)), _SMEM_SPEC],
          out_specs=(_rows(_DF), _t_spec()),
      )(*args, conv_basis[i + 1], conv_att[i + 1])
      t = t3.reshape(2 * _N, _D)
    else:
      out = pl.pallas_call(
          _update_head_body,
          grid=(_NV // _BLK,),
          out_shape=jax.ShapeDtypeStruct((_NV, 1), f32),
          in_specs=[_t_spec()] + [_rows(_DF)] * 4
                   + [_full((_DF, _DF)), _full((1, _DF)),
                      _full((5 * _DF, _D)), _full((1, _D)), _full((_D, _D)),
                      _full((1, _D)), _full((_D, _D)), _full((1, _D)),
                      _full((_D, 1)), _full((1, 1))],
          out_specs=_rows(1),
      )(partial, xs[0], xs[1], xs[2], x,
        conv_root[i], conv_bias[i].reshape(1, _DF),
        fc1_W, fc1_b.reshape(1, _D), fc2_W, fc2_b.reshape(1, _D),
        fc3_W, fc3_b.reshape(1, _D), fc4_W, fc4_b.reshape(1, 1))
    xs.append(x)

  return out.reshape(_NV)
